# Initial kernel scaffold; baseline (speedup 1.0000x reference)
import jax
import jax.numpy as jnp
from jax.experimental import pallas as pl


def kernel(coordinates, features, W1, b1, W2, b2, W3, b3, W4, b4, nW1, nb1, nW2, nb2):
    raise NotImplementedError("write your pallas kernel here")


# speedup vs baseline: 10.8900x; 10.8900x over previous
import functools

import jax
import jax.numpy as jnp
from jax import lax
from jax.experimental import pallas as pl
from jax.experimental.pallas import tpu as pltpu
from jax.experimental.pallas import tpu_sc as plsc

B = 8
N = 8192
M = 128
K = 16
F = 64
D = 768
LANES = 16
NCHUNK = N // LANES


def _fps_sc(xa, ya, za, ta):
    mesh = plsc.VectorSubcoreMesh(core_axis_name="c", subcore_axis_name="s")

    @functools.partial(
        pl.kernel,
        mesh=mesh,
        compiler_params=pltpu.CompilerParams(needs_layout_passes=False),
        out_type=jax.ShapeDtypeStruct((B, 4 * M), jnp.float32),
        scratch_types=[
            pltpu.VMEM((N,), jnp.float32),
            pltpu.VMEM((N,), jnp.float32),
            pltpu.VMEM((N,), jnp.float32),
            pltpu.VMEM((N,), jnp.float32),
            pltpu.VMEM((N,), jnp.float32),
            pltpu.VMEM((4 * M,), jnp.float32),
            pltpu.VMEM((32,), jnp.int32),
        ],
    )
    def k(x_hbm, y_hbm, z_hbm, t_hbm, cent_out, x_v, y_v, z_v, t_v, dist_v,
          cent_v, tmpi_v):
        wid = lax.axis_index("s") * 2 + lax.axis_index("c")

        @pl.when(wid < B)
        def _():
            b = wid
            pltpu.sync_copy(x_hbm.at[b], x_v)
            pltpu.sync_copy(y_hbm.at[b], y_v)
            pltpu.sync_copy(z_hbm.at[b], z_v)
            pltpu.sync_copy(t_hbm.at[b], t_v)

            lanes = lax.iota(jnp.int32, 16)
            inf16 = jnp.full((16,), jnp.inf, jnp.float32)
            sixteens = jnp.full((16,), 16, jnp.int32)

            def init_body(j, _):
                dist_v[pl.ds(j * 16, 16)] = inf16
                return 0

            lax.fori_loop(0, NCHUNK, init_body, 0, unroll=4)

            def bcast0_i(vec):
                tmpi_v[pl.ds(16, 16)] = vec
                return plsc.load_gather(tmpi_v, [sixteens])

            def bcast0_f(vec):
                return plsc.bitcast(
                    bcast0_i(plsc.bitcast(vec, jnp.int32)), jnp.float32)

            def coords_of(p_vec):
                return (plsc.load_gather(x_v, [p_vec]),
                        plsc.load_gather(y_v, [p_vec]),
                        plsc.load_gather(z_v, [p_vec]),
                        plsc.load_gather(t_v, [p_vec]))

            def store_cent(i, cx, cy, cz, ct):
                vals = jnp.where(
                    lanes == 0, cx,
                    jnp.where(lanes == 1, cy, jnp.where(lanes == 2, cz, ct)))
                safe = jnp.minimum(lanes, 3)
                plsc.store_scatter(cent_v, [i + M * safe], vals, mask=lanes < 4)

            c0 = (bcast0_f(x_v[pl.ds(0, 16)]), bcast0_f(y_v[pl.ds(0, 16)]),
                  bcast0_f(z_v[pl.ds(0, 16)]), bcast0_f(t_v[pl.ds(0, 16)]))
            store_cent(jnp.int32(0), *c0)

            def step(i, carry):
                cx, cy, cz, ct = carry

                def inner(j, ic):
                    bv, bi = ic
                    off = j * 16
                    dx = x_v[pl.ds(off, 16)] - cx
                    dy = y_v[pl.ds(off, 16)] - cy
                    dz = z_v[pl.ds(off, 16)] - cz
                    dt = t_v[pl.ds(off, 16)] - ct
                    d2 = dx * dx + dy * dy + dz * dz + dt * dt
                    nd = jnp.minimum(dist_v[pl.ds(off, 16)], d2)
                    dist_v[pl.ds(off, 16)] = nd
                    upd = nd > bv
                    bv = jnp.where(upd, nd, bv)
                    bi = jnp.where(upd, off + lanes, bi)
                    return bv, bi

                bv0 = jnp.full((16,), -1.0, jnp.float32)
                bi0 = jnp.zeros((16,), jnp.int32)
                bv, bi = lax.fori_loop(0, NCHUNK, inner, (bv0, bi0), unroll=4)
                sk, _ = plsc.sort_key_val(bv, bi, descending=True)
                hmax = bcast0_i(plsc.bitcast(sk, jnp.int32))
                hmaxf = plsc.bitcast(hmax, jnp.float32)
                cand = jnp.where(bv == hmaxf, bi, jnp.int32(1 << 30))
                sc_, _ = plsc.sort_key_val(cand, cand)
                p_vec = bcast0_i(sc_)
                nc = coords_of(p_vec)
                store_cent(i, *nc)
                return nc

            lax.fori_loop(1, M, step, c0)
            pltpu.sync_copy(cent_v, cent_out.at[b])

    return k(xa, ya, za, ta)


def _topk_tc(cent, ptsT):
    def body(cent_ref, pts_ref, knn_ref):
        b = pl.program_id(0)
        c = cent_ref[0]
        p = pts_ref[0]
        d2 = jnp.zeros((M, N), jnp.float32)
        for d in range(4):
            diff = c[:, d:d + 1] - p[d:d + 1, :]
            d2 = d2 + diff * diff
        iota = lax.broadcasted_iota(jnp.int32, (M, N), 1)
        big_i = jnp.int32(1 << 30)
        cols = []
        for _ in range(K):
            mn = jnp.min(d2, axis=1, keepdims=True)
            idx = jnp.min(jnp.where(d2 == mn, iota, big_i), axis=1)
            cols.append(idx)
            d2 = jnp.where(iota == idx[:, None], jnp.inf, d2)
        knn = jnp.stack(cols, axis=-1) + b * N
        knn_ref[...] = knn[None]

    return pl.pallas_call(
        body,
        grid=(B,),
        in_specs=[
            pl.BlockSpec((1, M, 4), lambda b: (b, 0, 0)),
            pl.BlockSpec((1, 4, N), lambda b: (b, 0, 0)),
        ],
        out_specs=pl.BlockSpec((1, M, K), lambda b: (b, 0, 0)),
        out_shape=jax.ShapeDtypeStruct((B, M, K), jnp.int32),
    )(cent, ptsT)


def _gather_sc(table, idx):
    nw = 32
    nb = B * M * K
    bpw = nb // nw
    fp = table.shape[1]
    mesh = plsc.VectorSubcoreMesh(core_axis_name="c", subcore_axis_name="s")

    @functools.partial(
        pl.kernel,
        mesh=mesh,
        compiler_params=pltpu.CompilerParams(needs_layout_passes=False),
        out_type=jax.ShapeDtypeStruct((nb, fp), jnp.float32),
        scratch_types=[
            pltpu.VMEM((bpw,), jnp.int32),
            pltpu.VMEM((bpw, fp), jnp.float32),
            pltpu.SemaphoreType.DMA,
        ],
    )
    def k(table_hbm, idx_hbm, out_hbm, idx_v, rows_v, sem):
        wid = lax.axis_index("s") * 2 + lax.axis_index("c")
        base = wid * bpw
        pltpu.sync_copy(idx_hbm.at[pl.ds(base, bpw)], idx_v)
        pltpu.async_copy(table_hbm.at[idx_v], rows_v, sem).wait()
        pltpu.sync_copy(rows_v, out_hbm.at[pl.ds(base, bpw)])

    return k(table, idx)


def _mlp_tc(x, W1, b1, W2, b2, W3, b3, W4, b4):
    rows = x.shape[0]
    fin = x.shape[1]
    blk = 2048
    grid = rows // blk

    def body(x_ref, w1, bb1, w2, bb2, w3, bb3, w4, bb4, out_ref):
        h = jnp.maximum(
            jnp.dot(x_ref[...], w1[...], preferred_element_type=jnp.float32)
            + bb1[...], 0.0)
        h = jnp.maximum(
            jnp.dot(h, w2[...], preferred_element_type=jnp.float32) + bb2[...], 0.0)
        h = jnp.maximum(
            jnp.dot(h, w3[...], preferred_element_type=jnp.float32) + bb3[...], 0.0)
        out_ref[...] = (
            jnp.dot(h, w4[...], preferred_element_type=jnp.float32) + bb4[...])

    full = lambda *s: pl.BlockSpec(s, lambda i: tuple(0 for _ in s))
    return pl.pallas_call(
        body,
        grid=(grid,),
        in_specs=[
            pl.BlockSpec((blk, fin), lambda i: (i, 0)),
            full(fin, 256), full(256,), full(256, 512), full(512,),
            full(512, D), full(D,), full(D, D), full(D,),
        ],
        out_specs=pl.BlockSpec((blk, D), lambda i: (i, 0)),
        out_shape=jax.ShapeDtypeStruct((rows, D), jnp.float32),
    )(x, W1, b1, W2, b2, W3, b3, W4, b4)


def _pool_tc(h, nW1, nb1, nW2, nb2):
    tokens = h.shape[0]
    blk = 128
    grid = tokens // blk

    def body(h_ref, w1, bb1, w2, bb2, out_ref):
        pooled = jnp.max(h_ref[...], axis=1)
        t1 = jnp.maximum(
            jnp.dot(pooled, w1[...], preferred_element_type=jnp.float32)
            + bb1[...], 0.0)
        out_ref[...] = (
            jnp.dot(t1, w2[...], preferred_element_type=jnp.float32) + bb2[...])

    full = lambda *s: pl.BlockSpec(s, lambda i: tuple(0 for _ in s))
    return pl.pallas_call(
        body,
        grid=(grid,),
        in_specs=[
            pl.BlockSpec((blk, K, D), lambda i: (i, 0, 0)),
            full(D, D), full(D,), full(D, D), full(D,),
        ],
        out_specs=pl.BlockSpec((blk, D), lambda i: (i, 0)),
        out_shape=jax.ShapeDtypeStruct((tokens, D), jnp.float32),
    )(h, nW1, nb1, nW2, nb2)


def kernel(coordinates, features, W1, b1, W2, b2, W3, b3, W4, b4, nW1, nb1, nW2, nb2):
    coords4 = coordinates[:, 1:5].reshape(B, N, 4).transpose(0, 2, 1)
    xa = coords4[:, 0, :]
    ya = coords4[:, 1, :]
    za = coords4[:, 2, :]
    ta = coords4[:, 3, :]

    cent_flat = _fps_sc(xa, ya, za, ta)
    cent = cent_flat.reshape(B, 4, M).transpose(0, 2, 1)
    knn = _topk_tc(cent, coords4)
    feats_p = jnp.concatenate(
        [features, jnp.zeros((B * N, 128 - F), jnp.float32)], axis=1)
    W1p = jnp.concatenate([W1, jnp.zeros((128 - F, 256), jnp.float32)], axis=0)
    gathered = _gather_sc(feats_p, knn.reshape(-1))
    h4 = _mlp_tc(gathered, W1p, b1, W2, b2, W3, b3, W4, b4)
    tokens = _pool_tc(h4.reshape(B * M, K, D), nW1, nb1, nW2, nb2)
    tokens = tokens.reshape(B, M, D)
    mask = jnp.ones((B, M), jnp.bool_)
    return tokens, cent, mask
